# baseline (speedup 1.0000x reference)
import functools

import jax
import jax.numpy as jnp
from jax import lax
from jax.experimental import pallas as pl
from jax.experimental.pallas import tpu as pltpu, tpu_sc as plsc

_ROWS = 100000
_NW = 32
_RPW = _ROWS // _NW
_CHUNK = 125
_NCHUNK = _RPW // _CHUNK

_mesh = plsc.VectorSubcoreMesh(core_axis_name="c", subcore_axis_name="s")


@functools.partial(
    pl.kernel,
    mesh=_mesh,
    out_type=(
        jax.ShapeDtypeStruct((_ROWS, 240), jnp.float32),
        jax.ShapeDtypeStruct((_ROWS, 240), jnp.float32),
    ),
    scratch_types=[
        pltpu.VMEM((_CHUNK, 480), jnp.float32),
        pltpu.VMEM((_CHUNK, 240), jnp.float32),
        pltpu.VMEM((_CHUNK, 240), jnp.float32),
        pltpu.SemaphoreType.DMA,
        pltpu.SemaphoreType.DMA,
    ],
    compiler_params=pltpu.CompilerParams(use_tc_tiling_on_sc=False),
)
def _half_split(x_hbm, out0_hbm, out1_hbm, ib, ob0, ob1, si, so):
    wid = lax.axis_index("s") * 2 + lax.axis_index("c")
    base = wid * _RPW

    def step(c, carry):
        r0 = base + c * _CHUNK
        din = pltpu.make_async_copy(
            x_hbm.at[pl.ds(r0, _CHUNK), :], ib, si)
        din.start()
        din.wait()
        d0 = pltpu.make_async_copy(ob0, out0_hbm.at[pl.ds(r0, _CHUNK), :], so)
        d1 = pltpu.make_async_copy(ob1, out1_hbm.at[pl.ds(r0, _CHUNK), :], so)
        d0.start()
        d1.start()
        d0.wait()
        d1.wait()
        return carry

    lax.fori_loop(0, _NCHUNK, step, 0)


def kernel(x):
    return _half_split(x)


# speedup vs baseline: 4.3848x; 4.3848x over previous
import functools

import jax
import jax.numpy as jnp
from jax.experimental import pallas as pl
from jax.experimental.pallas import tpu as pltpu

_ROWS = 100000
_BLOCK = 2000


def _body(x_ref, o0_ref, o1_ref):
    x = x_ref[...]
    o0_ref[...] = jnp.concatenate(
        [x[:, 0:64], x[:, 128:224], x[:, 320:400]], axis=1)
    o1_ref[...] = jnp.concatenate(
        [x[:, 64:128], x[:, 224:320], x[:, 400:480]], axis=1)


@jax.jit
def kernel(x):
    return pl.pallas_call(
        _body,
        grid=(_ROWS // _BLOCK,),
        in_specs=[pl.BlockSpec((_BLOCK, 480), lambda i: (i, 0))],
        out_specs=(
            pl.BlockSpec((_BLOCK, 240), lambda i: (i, 0)),
            pl.BlockSpec((_BLOCK, 240), lambda i: (i, 0)),
        ),
        out_shape=(
            jax.ShapeDtypeStruct((_ROWS, 240), jnp.float32),
            jax.ShapeDtypeStruct((_ROWS, 240), jnp.float32),
        ),
        compiler_params=pltpu.CompilerParams(
            dimension_semantics=("arbitrary",),
        ),
    )(x)


# speedup vs baseline: 4.3871x; 1.0005x over previous
import jax
import jax.numpy as jnp
from jax.experimental import pallas as pl
from jax.experimental.pallas import tpu as pltpu

_ROWS = 100000
_BLOCK = 2000


def _body(x_ref, o0_ref, o1_ref):
    x = x_ref[...]
    o0_ref[...] = jnp.concatenate(
        [x[:, 0:64], x[:, 128:224], x[:, 320:400]], axis=1)
    o1_ref[...] = jnp.concatenate(
        [x[:, 64:128], x[:, 224:320], x[:, 400:480]], axis=1)


@jax.jit
def kernel(x):
    return pl.pallas_call(
        _body,
        grid=(_ROWS // _BLOCK,),
        in_specs=[pl.BlockSpec((_BLOCK, 480), lambda i: (i, 0))],
        out_specs=(
            pl.BlockSpec((_BLOCK, 240), lambda i: (i, 0)),
            pl.BlockSpec((_BLOCK, 240), lambda i: (i, 0)),
        ),
        out_shape=(
            jax.ShapeDtypeStruct((_ROWS, 240), jnp.float32),
            jax.ShapeDtypeStruct((_ROWS, 240), jnp.float32),
        ),
        compiler_params=pltpu.CompilerParams(
            dimension_semantics=("arbitrary",),
        ),
    )(x)


# speedup vs baseline: 4.3927x; 1.0013x over previous
import jax
import jax.numpy as jnp
from jax.experimental import pallas as pl
from jax.experimental.pallas import tpu as pltpu

_ROWS = 100000
_BLOCK = 2000


def _body(x_ref, o0_ref, o1_ref):
    x = x_ref[...]
    o0_ref[...] = jnp.concatenate(
        [x[:, 0:64], x[:, 128:224], x[:, 320:400]], axis=1)
    o1_ref[...] = jnp.concatenate(
        [x[:, 64:128], x[:, 224:320], x[:, 400:480]], axis=1)


@jax.jit
def kernel(x):
    return pl.pallas_call(
        _body,
        grid=(_ROWS // _BLOCK,),
        in_specs=[pl.BlockSpec((_BLOCK, 480), lambda i: (i, 0))],
        out_specs=(
            pl.BlockSpec((_BLOCK, 240), lambda i: (i, 0)),
            pl.BlockSpec((_BLOCK, 240), lambda i: (i, 0)),
        ),
        out_shape=(
            jax.ShapeDtypeStruct((_ROWS, 240), jnp.float32),
            jax.ShapeDtypeStruct((_ROWS, 240), jnp.float32),
        ),
        compiler_params=pltpu.CompilerParams(
            dimension_semantics=("parallel",),
        ),
    )(x)


# speedup vs baseline: 4.4391x; 1.0106x over previous
import jax
import jax.numpy as jnp
from jax.experimental import pallas as pl
from jax.experimental.pallas import tpu as pltpu

_ROWS = 100000
_BLOCK = 4000


def _body(x_ref, o0_ref, o1_ref):
    x = x_ref[...]
    o0_ref[...] = jnp.concatenate(
        [x[:, 0:64], x[:, 128:224], x[:, 320:400]], axis=1)
    o1_ref[...] = jnp.concatenate(
        [x[:, 64:128], x[:, 224:320], x[:, 400:480]], axis=1)


@jax.jit
def kernel(x):
    return pl.pallas_call(
        _body,
        grid=(_ROWS // _BLOCK,),
        in_specs=[pl.BlockSpec((_BLOCK, 480), lambda i: (i, 0))],
        out_specs=(
            pl.BlockSpec((_BLOCK, 240), lambda i: (i, 0)),
            pl.BlockSpec((_BLOCK, 240), lambda i: (i, 0)),
        ),
        out_shape=(
            jax.ShapeDtypeStruct((_ROWS, 240), jnp.float32),
            jax.ShapeDtypeStruct((_ROWS, 240), jnp.float32),
        ),
        compiler_params=pltpu.CompilerParams(
            dimension_semantics=("parallel",),
        ),
    )(x)
